# baseline (speedup 1.0000x reference)
import functools

import jax
import jax.numpy as jnp
from jax import lax
from jax.experimental import pallas as pl
from jax.experimental.pallas import tpu as pltpu
from jax.experimental.pallas import tpu_sc as plsc

_N = 10000
_D = 128
_E = 320000
_NC = 2
_NS = 16
_NW = _NC * _NS
_L = 16
_C = 128
_C0 = 128
_C1 = 32
_E_PAD = (_C0 + _C1) * _NS * _C
_CORE0_EDGES = _NS * _C0 * _C
_N_PAD = 10112
_STRIPE = _N_PAD // _NS


def _make_segsum(with_deg):
  mesh = plsc.VectorSubcoreMesh(core_axis_name="c", subcore_axis_name="s")
  out_type = [jax.ShapeDtypeStruct((_NC, _N_PAD, _D), jnp.float32)]
  scratch = [
      pltpu.VMEM((_C,), jnp.int32),
      pltpu.VMEM((_C,), jnp.int32),
      pltpu.VMEM((_C, _D), jnp.float32),
      pltpu.VMEM_SHARED((_N_PAD, _D), jnp.float32),
      pltpu.SemaphoreType.DMA,
  ]
  if with_deg:
    out_type.append(jax.ShapeDtypeStruct((_NW, 1, _N_PAD), jnp.float32))
    scratch.append(pltpu.VMEM((_N_PAD,), jnp.float32))

  def body(x_hbm, src_hbm, dst_hbm, zeros_hbm, *refs):
    if with_deg:
      acc_out, deg_out, src_v, dst_v, rows_v, acc_sh, sem, deg_v = refs
    else:
      acc_out, src_v, dst_v, rows_v, acc_sh, sem = refs
    c = lax.axis_index("c")
    s = lax.axis_index("s")
    wid = s * _NC + c
    r0 = s * _STRIPE
    pltpu.sync_copy(zeros_hbm.at[pl.ds(r0, _STRIPE)],
                    acc_sh.at[pl.ds(r0, _STRIPE)])
    if with_deg:
      z16 = jnp.zeros((_L,), jnp.float32)

      def zero_deg(i, carry):
        deg_v[pl.ds(i * _L, _L)] = z16
        return carry

      lax.fori_loop(0, _N_PAD // _L, zero_deg, 0)
    plsc.subcore_barrier()

    ones16 = jnp.ones((_L,), jnp.float32)

    def chunk_body(ebase):
      def chunk(i, carry):
        off = ebase + i * _C
        pltpu.sync_copy(src_hbm.at[pl.ds(off, _C)], src_v)
        pltpu.sync_copy(dst_hbm.at[pl.ds(off, _C)], dst_v)
        pltpu.async_copy(x_hbm.at[src_v], rows_v, sem).wait()
        pltpu.sync_copy(rows_v, acc_sh.at[dst_v], add=True)
        if with_deg:
          for l in range(_C // _L):
            plsc.addupdate_scatter(deg_v, [dst_v[pl.ds(l * _L, _L)]], ones16)
        return carry

      return chunk

    @pl.when(c == 0)
    def _():
      lax.fori_loop(0, _C0, chunk_body(s * ((_C0 + _C1) * _C)), 0)

    @pl.when(c == 1)
    def _():
      lax.fori_loop(0, _C1,
                    chunk_body(s * ((_C0 + _C1) * _C) + _C0 * _C), 0)

    plsc.subcore_barrier()
    pltpu.sync_copy(acc_sh.at[pl.ds(r0, _STRIPE)],
                    acc_out.at[c, pl.ds(r0, _STRIPE)])
    if with_deg:
      pltpu.sync_copy(deg_v, deg_out.at[wid, 0])

  out = out_type if with_deg else out_type[0]
  params = pltpu.CompilerParams(needs_layout_passes=False) if with_deg else None
  return pl.kernel(body, out_type=out, mesh=mesh, scratch_types=scratch,
                   compiler_params=params)


_segsum_deg = _make_segsum(True)
_segsum = _make_segsum(False)

_BN = 1000


def _dense_body(relu, p_ref, d_ref, x_ref, wl_ref, wr_ref, b_ref, o_ref):
  p = p_ref[...]
  deg = jnp.sum(d_ref[...], axis=1, keepdims=True)
  agg = (p[0] + p[1]) / jnp.maximum(deg, 1.0)
  out = lax.dot_general(agg, wl_ref[...], (((1,), (1,)), ((), ())),
                        preferred_element_type=jnp.float32)
  out = out + lax.dot_general(x_ref[...], wr_ref[...], (((1,), (1,)), ((), ())),
                              preferred_element_type=jnp.float32)
  out = out + b_ref[...]
  if relu:
    out = jnp.maximum(out, 0.0)
  o_ref[...] = out


def _dense(parts, degT, xin, Wl, Wr, b, relu):
  return pl.pallas_call(
      functools.partial(_dense_body, relu),
      grid=(_N // _BN,),
      in_specs=[
          pl.BlockSpec((_NC, _BN, _D), lambda i: (0, i, 0)),
          pl.BlockSpec((_BN, _NW), lambda i: (i, 0)),
          pl.BlockSpec((_BN, _D), lambda i: (i, 0)),
          pl.BlockSpec((_D, _D), lambda i: (0, 0)),
          pl.BlockSpec((_D, _D), lambda i: (0, 0)),
          pl.BlockSpec((1, _D), lambda i: (0, 0)),
      ],
      out_specs=pl.BlockSpec((_BN, _D), lambda i: (i, 0)),
      out_shape=jax.ShapeDtypeStruct((_N, _D), jnp.float32),
  )(parts, degT, xin, Wl, Wr, b.reshape(1, _D))


def kernel(x, edge_index, W1l, b1, W1r, W2l, b2, W2r):
  pad = _E_PAD - _E
  pad_dst = _N + (jnp.arange(pad, dtype=jnp.int32) % (_N_PAD - _N))
  src = jnp.concatenate([edge_index[0], jnp.zeros((pad,), jnp.int32)])
  dst = jnp.concatenate([edge_index[1], pad_dst])
  zeros = jnp.zeros((_N_PAD, _D), jnp.float32)
  parts1, deg32 = _segsum_deg(x, src, dst, zeros)
  degT = deg32.reshape(_NW, _N_PAD).T
  h = _dense(parts1, degT, x, W1l, W1r, b1, True)
  parts2 = _segsum(h, src, dst, zeros)
  return _dense(parts2, degT, h, W2l, W2r, b2, False)


# speedup vs baseline: 1.3492x; 1.3492x over previous
import functools

import jax
import jax.numpy as jnp
from jax import lax
from jax.experimental import pallas as pl
from jax.experimental.pallas import tpu as pltpu
from jax.experimental.pallas import tpu_sc as plsc

_N = 10000
_D = 128
_E = 320000
_NC = 2
_NS = 16
_NW = _NC * _NS
_L = 16
_C = 128
_CHUNKS = -(-_E // (_NW * _C))
_PER_TILE = _CHUNKS * _C
_E_PAD = _PER_TILE * _NW
_N_PAD = 10112
_STRIPE = _N_PAD // _NS


def _make_segsum(with_deg):
  mesh = plsc.VectorSubcoreMesh(core_axis_name="c", subcore_axis_name="s")
  out_type = [jax.ShapeDtypeStruct((_NC, _N_PAD, _D), jnp.float32)]
  scratch = [
      pltpu.VMEM((_C,), jnp.int32),
      pltpu.VMEM((_C,), jnp.int32),
      pltpu.VMEM((_C, _D), jnp.float32),
      pltpu.VMEM_SHARED((_N_PAD, _D), jnp.float32),
      pltpu.SemaphoreType.DMA,
  ]
  if with_deg:
    out_type.append(jax.ShapeDtypeStruct((_NW, 1, _N_PAD), jnp.float32))
    scratch.append(pltpu.VMEM((_N_PAD,), jnp.float32))

  def body(x_hbm, src_hbm, dst_hbm, zeros_hbm, *refs):
    if with_deg:
      acc_out, deg_out, src_v, dst_v, rows_v, acc_sh, sem, deg_v = refs
    else:
      acc_out, src_v, dst_v, rows_v, acc_sh, sem = refs
    c = lax.axis_index("c")
    s = lax.axis_index("s")
    wid = s * _NC + c
    r0 = s * _STRIPE
    pltpu.sync_copy(zeros_hbm.at[pl.ds(r0, _STRIPE)],
                    acc_sh.at[pl.ds(r0, _STRIPE)])
    if with_deg:
      z16 = jnp.zeros((_L,), jnp.float32)

      def zero_deg(i, carry):
        deg_v[pl.ds(i * _L, _L)] = z16
        return carry

      lax.fori_loop(0, _N_PAD // _L, zero_deg, 0)
    plsc.subcore_barrier()

    ones16 = jnp.ones((_L,), jnp.float32)

    def chunk_body(ebase):
      def chunk(i, carry):
        off = ebase + i * _C
        pltpu.sync_copy(src_hbm.at[pl.ds(off, _C)], src_v)
        pltpu.sync_copy(dst_hbm.at[pl.ds(off, _C)], dst_v)
        pltpu.async_copy(x_hbm.at[src_v], rows_v, sem).wait()
        pltpu.sync_copy(rows_v, acc_sh.at[dst_v], add=True)
        if with_deg:
          for l in range(_C // _L):
            plsc.addupdate_scatter(deg_v, [dst_v[pl.ds(l * _L, _L)]], ones16)
        return carry

      return chunk

    lax.fori_loop(0, _CHUNKS, chunk_body(wid * _PER_TILE), 0)
    plsc.subcore_barrier()
    pltpu.sync_copy(acc_sh.at[pl.ds(r0, _STRIPE)],
                    acc_out.at[c, pl.ds(r0, _STRIPE)])
    if with_deg:
      pltpu.sync_copy(deg_v, deg_out.at[wid, 0])

  out = out_type if with_deg else out_type[0]
  params = pltpu.CompilerParams(needs_layout_passes=False) if with_deg else None
  return pl.kernel(body, out_type=out, mesh=mesh, scratch_types=scratch,
                   compiler_params=params)


_segsum_deg = _make_segsum(True)
_segsum = _make_segsum(False)

_BN = 1000


def _dense_body(relu, p_ref, d_ref, x_ref, wl_ref, wr_ref, b_ref, o_ref):
  p = p_ref[...]
  deg = jnp.sum(d_ref[...], axis=1, keepdims=True)
  agg = (p[0] + p[1]) / jnp.maximum(deg, 1.0)
  out = lax.dot_general(agg, wl_ref[...], (((1,), (1,)), ((), ())),
                        preferred_element_type=jnp.float32)
  out = out + lax.dot_general(x_ref[...], wr_ref[...], (((1,), (1,)), ((), ())),
                              preferred_element_type=jnp.float32)
  out = out + b_ref[...]
  if relu:
    out = jnp.maximum(out, 0.0)
  o_ref[...] = out


def _dense(parts, degT, xin, Wl, Wr, b, relu):
  return pl.pallas_call(
      functools.partial(_dense_body, relu),
      grid=(_N // _BN,),
      in_specs=[
          pl.BlockSpec((_NC, _BN, _D), lambda i: (0, i, 0)),
          pl.BlockSpec((_BN, _NW), lambda i: (i, 0)),
          pl.BlockSpec((_BN, _D), lambda i: (i, 0)),
          pl.BlockSpec((_D, _D), lambda i: (0, 0)),
          pl.BlockSpec((_D, _D), lambda i: (0, 0)),
          pl.BlockSpec((1, _D), lambda i: (0, 0)),
      ],
      out_specs=pl.BlockSpec((_BN, _D), lambda i: (i, 0)),
      out_shape=jax.ShapeDtypeStruct((_N, _D), jnp.float32),
  )(parts, degT, xin, Wl, Wr, b.reshape(1, _D))


def kernel(x, edge_index, W1l, b1, W1r, W2l, b2, W2r):
  pad = _E_PAD - _E
  pad_dst = _N + (jnp.arange(pad, dtype=jnp.int32) % (_N_PAD - _N))
  src = jnp.concatenate([edge_index[0], jnp.zeros((pad,), jnp.int32)])
  dst = jnp.concatenate([edge_index[1], pad_dst])
  zeros = jnp.zeros((_N_PAD, _D), jnp.float32)
  parts1, deg32 = _segsum_deg(x, src, dst, zeros)
  degT = deg32.reshape(_NW, _N_PAD).T
  h = _dense(parts1, degT, x, W1l, W1r, b1, True)
  parts2 = _segsum(h, src, dst, zeros)
  return _dense(parts2, degT, h, W2l, W2r, b2, False)
